# baseline (speedup 1.0000x reference)
import functools

import jax
import jax.numpy as jnp
from jax import lax
from jax.experimental import pallas as pl
from jax.experimental.pallas import tpu as pltpu
from jax.experimental.pallas import tpu_sc as plsc

N = 10000
E = 160000
FD = 256
NP = 10240
EP = 163840
RPT = NP // 32
EC = 1280
NCH = EP // EC
GB = 56
CAP = EC + GB + 64


def _sc_spmm(xp, srcp, dstp, ewp, prevp):
    mesh = plsc.VectorSubcoreMesh(core_axis_name="c", subcore_axis_name="s")

    @functools.partial(
        pl.kernel,
        out_type=jax.ShapeDtypeStruct((NP, FD), jnp.float32),
        mesh=mesh,
        compiler_params=pltpu.CompilerParams(needs_layout_passes=False),
        scratch_types=[
            pltpu.VMEM((RPT, FD), jnp.float32),
            [pltpu.VMEM((EC,), jnp.int32)] * 2,
            [pltpu.VMEM((EC,), jnp.int32)] * 2,
            [pltpu.VMEM((EC,), jnp.float32)] * 2,
            pltpu.VMEM((CAP,), jnp.int32),
            pltpu.VMEM((CAP,), jnp.int32),
            pltpu.VMEM((CAP,), jnp.float32),
            pltpu.VMEM((GB, FD), jnp.float32),
            pltpu.SemaphoreType.DMA,
            [pltpu.SemaphoreType.DMA] * 2,
        ],
    )
    def k(xp_hbm, src_hbm, dst_hbm, ew_hbm, prev_hbm, h_hbm,
          acc, src_b, dst_b, ew_b, msrc, mloc, mew, rbuf, gsem, ssems):
        c = lax.axis_index("c")
        s = lax.axis_index("s")
        t = c * 16 + s
        lo = t * RPT
        hi = lo + RPT

        pltpu.sync_copy(prev_hbm.at[pl.ds(lo, RPT)], mloc.at[pl.ds(0, RPT)])
        d0 = pltpu.async_copy(xp_hbm.at[mloc.at[pl.ds(0, 128)]],
                              acc.at[pl.ds(0, 128)], gsem)
        d1 = pltpu.async_copy(xp_hbm.at[mloc.at[pl.ds(128, 128)]],
                              acc.at[pl.ds(128, 128)], gsem)
        d2 = pltpu.async_copy(xp_hbm.at[mloc.at[pl.ds(256, 64)]],
                              acc.at[pl.ds(256, 64)], gsem)
        d0.wait()
        d1.wait()
        d2.wait()

        def stage(ch, st):
            e0 = ch * EC
            pltpu.async_copy(src_hbm.at[pl.ds(e0, EC)], src_b[st], ssems[st])
            pltpu.async_copy(dst_hbm.at[pl.ds(e0, EC)], dst_b[st], ssems[st])
            pltpu.async_copy(ew_hbm.at[pl.ds(e0, EC)], ew_b[st], ssems[st])

        def stage_wait(ch, st):
            e0 = ch * EC
            pltpu.make_async_copy(src_hbm.at[pl.ds(e0, EC)], src_b[st],
                                  ssems[st]).wait()
            pltpu.make_async_copy(dst_hbm.at[pl.ds(e0, EC)], dst_b[st],
                                  ssems[st]).wait()
            pltpu.make_async_copy(ew_hbm.at[pl.ds(e0, EC)], ew_b[st],
                                  ssems[st]).wait()

        def do_batch(base):
            gd = pltpu.async_copy(xp_hbm.at[msrc.at[pl.ds(base, GB)]],
                                  rbuf, gsem)
            gd.wait()

            def acc_edge(j, _):
                e = base + j
                wv = mew[pl.ds(e, 16)]
                w16 = jnp.full((16,), wv[0])
                rv = mloc[pl.ds(e, 16)]
                r = rv[0]
                for k16 in range(FD // 16):
                    sl = pl.ds(k16 * 16, 16)
                    acc[r, sl] = acc[r, sl] + rbuf[j, sl] * w16
                return 0

            lax.fori_loop(0, GB, acc_edge, 0)

        def process_chunk(ch, st, cnt):
            stage_wait(ch, st)

            def scan_step(v, cnt):
                off = v * 16
                dv = dst_b[st][pl.ds(off, 16)]
                sv = src_b[st][pl.ds(off, 16)]
                wv = ew_b[st][pl.ds(off, 16)]
                m = (dv >= lo) & (dv < hi)
                plsc.store_compressed(msrc.at[pl.ds(cnt, 16)], sv, mask=m)
                plsc.store_compressed(mloc.at[pl.ds(cnt, 16)], dv - lo, mask=m)
                plsc.store_compressed(mew.at[pl.ds(cnt, 16)], wv, mask=m)
                npop = plsc.all_reduce_population_count(m)
                return cnt + npop[0]

            cnt = lax.fori_loop(0, EC // 16, scan_step, cnt)
            nb = cnt // GB

            def batch_body(bi, _):
                do_batch(bi * GB)
                return 0

            lax.fori_loop(0, nb, batch_body, 0)
            rem = cnt - nb * GB

            @pl.when(nb > 0)
            def _():
                for i in range(4):
                    sl_d = pl.ds(i * 16, 16)
                    sl_s = pl.ds(nb * GB + i * 16, 16)
                    msrc[sl_d] = msrc[sl_s]
                    mloc[sl_d] = mloc[sl_s]
                    mew[sl_d] = mew[sl_s]

            return rem

        stage(0, 0)

        def chunk_pair(g, cnt):
            stage(2 * g + 1, 1)
            cnt = process_chunk(2 * g, 0, cnt)

            @pl.when(g < NCH // 2 - 1)
            def _():
                stage(2 * g + 2, 0)

            cnt = process_chunk(2 * g + 1, 1, cnt)
            return cnt

        rem = lax.fori_loop(0, NCH // 2, chunk_pair, jnp.int32(0))

        z16 = jnp.full((16,), rem * 0)
        zf16 = jnp.full((16,), (rem * 0).astype(jnp.float32))
        for i in range(4):
            sl = pl.ds(rem + i * 16, 16)
            msrc[sl] = z16
            mloc[sl] = z16
            mew[sl] = zf16

        @pl.when(rem > 0)
        def _():
            do_batch(0)

        pltpu.sync_copy(acc, h_hbm.at[pl.ds(lo, RPT), :])

    return k(xp, srcp, dstp, ewp, prevp)


def _tc_head(h, fs_w2, w2, b2, w_out, b_out):
    blk = 1280
    grid = NP // blk

    def body(h_ref, fsw_ref, w2_ref, b2_ref, wout_ref, bout_ref, out_ref):
        g = jax.nn.sigmoid(fsw_ref[...]) * 0.5
        h2 = h_ref[...] * g
        acc = jnp.dot(h2, w2_ref[...], preferred_element_type=jnp.float32) \
            + b2_ref[...]
        x = jnp.maximum(acc, 0.0)
        out_ref[...] = (
            jnp.dot(x, wout_ref[...], preferred_element_type=jnp.float32)
            + bout_ref[...]
        )

    return pl.pallas_call(
        body,
        grid=(grid,),
        in_specs=[
            pl.BlockSpec((blk, FD), lambda i: (i, 0)),
            pl.BlockSpec((1, FD), lambda i: (0, 0)),
            pl.BlockSpec((FD, 128), lambda i: (0, 0)),
            pl.BlockSpec((1, 128), lambda i: (0, 0)),
            pl.BlockSpec((128, 40), lambda i: (0, 0)),
            pl.BlockSpec((1, 40), lambda i: (0, 0)),
        ],
        out_specs=pl.BlockSpec((blk, 40), lambda i: (i, 0)),
        out_shape=jax.ShapeDtypeStruct((NP, 40), jnp.float32),
    )(h, fs_w2, w2, b2, w_out, b_out)


def kernel(X, edge_index, edge_weight, previous_indices, sampled_nodes,
           fs_w, W1, b1, W2, b2, W_out, b_out):
    src = edge_index[2, 0]
    dst = edge_index[2, 1]
    ew = edge_weight[2]
    prev = previous_indices[2]

    pad_rows = NP - N
    xp = jnp.pad(X, ((0, pad_rows), (0, 0)))
    srcp = jnp.pad(src, (0, EP - E))
    dstp = jnp.pad(dst, (0, EP - E))
    ewp = jnp.pad(ew, (0, EP - E))
    prevp = jnp.pad(prev, (0, pad_rows))

    h = _sc_spmm(xp, srcp, dstp, ewp, prevp)

    out = _tc_head(h, fs_w.reshape(1, FD), W2, b2.reshape(1, -1),
                   W_out, b_out.reshape(1, -1))
    return out[:N]


# speedup vs baseline: 3.6635x; 3.6635x over previous
import functools

import jax
import jax.numpy as jnp
from jax import lax
from jax.experimental import pallas as pl
from jax.experimental.pallas import tpu as pltpu
from jax.experimental.pallas import tpu_sc as plsc

N = 10000
E = 160000
D = 128
NP = 10240
EP = 163840
B = 128
EDGES_PER_TILE = EP // 16
BATCHES = EDGES_PER_TILE // B
ROWS_PER_TILE = NP // 16
ROW_CHUNKS = ROWS_PER_TILE // B
NBUF = 2
CB = 8


def _sc_spmm(xc, srcp, dstp2, ewp, prevp):
    mesh = plsc.VectorSubcoreMesh(core_axis_name="c", subcore_axis_name="s")
    bpt = BATCHES

    @functools.partial(
        pl.kernel,
        out_type=jax.ShapeDtypeStruct((2 * NP, D), jnp.float32),
        mesh=mesh,
        scratch_types=[
            pltpu.VMEM_SHARED((NP, D), jnp.float32),
            [pltpu.VMEM((CB * B,), jnp.int32)] * 2,
            [pltpu.VMEM((CB, B), jnp.int32)] * 2,
            [pltpu.VMEM((CB * B + 16,), jnp.float32)] * 2,
            [pltpu.VMEM((B, D), jnp.float32)] * NBUF,
            [pltpu.SemaphoreType.DMA] * NBUF,
            [pltpu.SemaphoreType.DMA] * 2,
        ],
    )
    def k(xc_hbm, src_hbm, dst_hbm, ew_hbm, prev_hbm, h_hbm,
          s_acc, idx_b, dst_b, ew_b, rows, sems, ssems):
        idx_a = idx_b[0]
        c = lax.axis_index("c")
        s = lax.axis_index("s")
        coff = c * NP

        def scale_rows(rbuf, ew_a, e_lo):
            @plsc.parallel_loop(0, B, 1, unroll=4)
            def _(j):
                wv = ew_a[pl.ds(e_lo + j, 16)]
                w16 = jnp.full((16,), wv[0])
                for k8 in range(D // 16):
                    sl = (j, pl.ds(k8 * 16, 16))
                    rbuf[sl] = rbuf[sl] * w16

        pltpu.sync_copy(prev_hbm.at[pl.ds(s * ROWS_PER_TILE, ROWS_PER_TILE)],
                        idx_a.at[pl.ds(0, ROWS_PER_TILE)])

        def add_prev_off(j, _):
            sl = pl.ds(j * 16, 16)
            idx_a[sl] = idx_a[sl] + coff
            return 0

        lax.fori_loop(0, ROWS_PER_TILE // 16, add_prev_off, 0)
        row_base = s * ROWS_PER_TILE
        g1 = [None] * ROW_CHUNKS
        for j in range(ROW_CHUNKS):
            g1[j] = pltpu.async_copy(xc_hbm.at[idx_a.at[pl.ds(j * B, B)]],
                                     rows[j % NBUF], sems[j % NBUF])
            if j >= 1:
                g1[j - 1].wait()
                pltpu.sync_copy(rows[(j - 1) % NBUF],
                                s_acc.at[pl.ds(row_base + (j - 1) * B, B)])
        g1[ROW_CHUNKS - 1].wait()
        pltpu.sync_copy(rows[(ROW_CHUNKS - 1) % NBUF],
                        s_acc.at[pl.ds(row_base + (ROW_CHUNKS - 1) * B, B)])
        plsc.subcore_barrier()

        NCH = bpt // CB

        def stage(ch, st):
            e_base = s * EDGES_PER_TILE + ch * (CB * B)
            pltpu.async_copy(src_hbm.at[pl.ds(e_base, CB * B)], idx_b[st],
                             ssems[st])
            pltpu.async_copy(dst_hbm.at[pl.ds(s * bpt + ch * CB, CB), :],
                             dst_b[st], ssems[st])
            pltpu.async_copy(ew_hbm.at[pl.ds(e_base, CB * B)],
                             ew_b[st].at[pl.ds(0, CB * B)], ssems[st])

        def stage_wait(ch, st):
            e_base = s * EDGES_PER_TILE + ch * (CB * B)
            pltpu.make_async_copy(src_hbm.at[pl.ds(e_base, CB * B)], idx_b[st],
                                  ssems[st]).wait()
            pltpu.make_async_copy(dst_hbm.at[pl.ds(s * bpt + ch * CB, CB), :],
                                  dst_b[st], ssems[st]).wait()
            pltpu.make_async_copy(ew_hbm.at[pl.ds(e_base, CB * B)],
                                  ew_b[st].at[pl.ds(0, CB * B)],
                                  ssems[st]).wait()

        def process_chunk(ch, st):
            stage_wait(ch, st)

            @plsc.parallel_loop(0, CB * B // 16, 1, unroll=4)
            def _(b2):
                sl = pl.ds(b2 * 16, 16)
                idx_b[st][sl] = idx_b[st][sl] + coff

            gd = [None] * CB
            sd = [None] * CB

            def do_batch(b):
                buf = b % NBUF
                gd[b].wait()
                scale_rows(rows[buf], ew_b[st], b * B)
                sd[b] = pltpu.async_copy(rows[buf], s_acc.at[dst_b[st].at[b]],
                                         sems[buf], add=True)

            for b in range(CB):
                buf = b % NBUF
                if b >= NBUF:
                    sd[b - NBUF].wait()
                gd[b] = pltpu.async_copy(
                    xc_hbm.at[idx_b[st].at[pl.ds(b * B, B)]], rows[buf],
                    sems[buf])
                if b >= 1:
                    do_batch(b - 1)
            do_batch(CB - 1)
            for t in range(max(0, CB - NBUF), CB):
                sd[t].wait()

        stage(0, 0)

        def chunk_pair(g, _):
            stage(2 * g + 1, 1)
            process_chunk(2 * g, 0)

            @pl.when(g < NCH // 2 - 1)
            def _():
                stage(2 * g + 2, 0)

            process_chunk(2 * g + 1, 1)
            return 0

        lax.fori_loop(0, NCH // 2, chunk_pair, 0)
        plsc.subcore_barrier()

        rd = [None] * ROW_CHUNKS
        wr = [None] * ROW_CHUNKS
        for j in range(ROW_CHUNKS):
            buf = j % NBUF
            if j >= NBUF:
                wr[j - NBUF].wait()
            rd[j] = pltpu.async_copy(s_acc.at[pl.ds(row_base + j * B, B)],
                                     rows[buf], sems[buf])
            rd[j].wait()
            wr[j] = pltpu.async_copy(
                rows[buf], h_hbm.at[pl.ds(coff + row_base + j * B, B)],
                sems[buf])
        for j in range(ROW_CHUNKS - NBUF, ROW_CHUNKS):
            wr[j].wait()

    return k(xc, srcp, dstp2, ewp, prevp)


def _tc_head(h, fs_w2, w2, b2, w_out, b_out):
    blk = 1280
    grid = NP // blk
    hi_blocks = NP // blk

    def body(hlo_ref, hhi_ref, fsw_ref, w2_ref, b2_ref, wout_ref, bout_ref, out_ref):
        g = jax.nn.sigmoid(fsw_ref[...]) * 0.5
        hlo = hlo_ref[...] * g[:, :D]
        hhi = hhi_ref[...] * g[:, D:]
        acc = (
            jnp.dot(hlo, w2_ref[:D, :], preferred_element_type=jnp.float32)
            + jnp.dot(hhi, w2_ref[D:, :], preferred_element_type=jnp.float32)
            + b2_ref[...]
        )
        x = jnp.maximum(acc, 0.0)
        out_ref[...] = (
            jnp.dot(x, wout_ref[...], preferred_element_type=jnp.float32)
            + bout_ref[...]
        )

    return pl.pallas_call(
        body,
        grid=(grid,),
        in_specs=[
            pl.BlockSpec((blk, D), lambda i: (i, 0)),
            pl.BlockSpec((blk, D), lambda i: (i + hi_blocks, 0)),
            pl.BlockSpec((1, 2 * D), lambda i: (0, 0)),
            pl.BlockSpec((2 * D, D), lambda i: (0, 0)),
            pl.BlockSpec((1, D), lambda i: (0, 0)),
            pl.BlockSpec((D, 40), lambda i: (0, 0)),
            pl.BlockSpec((1, 40), lambda i: (0, 0)),
        ],
        out_specs=pl.BlockSpec((blk, 40), lambda i: (i, 0)),
        out_shape=jax.ShapeDtypeStruct((NP, 40), jnp.float32),
    )(h, h, fs_w2, w2, b2, w_out, b_out)


def kernel(X, edge_index, edge_weight, previous_indices, sampled_nodes,
           fs_w, W1, b1, W2, b2, W_out, b_out):
    src = edge_index[2, 0]
    dst = edge_index[2, 1]
    ew = edge_weight[2]
    prev = previous_indices[2]

    pad_rows = NP - N
    xlo = jnp.pad(X[:, :D], ((0, pad_rows), (0, 0)))
    xhi = jnp.pad(X[:, D:], ((0, pad_rows), (0, 0)))
    xc = jnp.concatenate([xlo, xhi], axis=0)
    srcp = jnp.pad(src, (0, EP - E))
    dstp2 = jnp.pad(dst, (0, EP - E)).reshape(EP // B, B)
    ewp = jnp.pad(ew, (0, EP - E))
    prevp = jnp.pad(prev, (0, pad_rows))

    h = _sc_spmm(xc, srcp, dstp2, ewp, prevp)

    out = _tc_head(h, fs_w.reshape(1, 2 * D), W2, b2.reshape(1, -1),
                   W_out, b_out.reshape(1, -1))
    return out[:N]


# speedup vs baseline: 3.7517x; 1.0241x over previous
import functools

import jax
import jax.numpy as jnp
from jax import lax
from jax.experimental import pallas as pl
from jax.experimental.pallas import tpu as pltpu
from jax.experimental.pallas import tpu_sc as plsc

N = 10000
E = 160000
D = 128
NP = 10240
EP = 163840
B = 128
EDGES_PER_TILE = EP // 16
BATCHES = EDGES_PER_TILE // B
ROWS_PER_TILE = NP // 16
ROW_CHUNKS = ROWS_PER_TILE // B
NBUF = 2
CB = 16


def _sc_spmm(xc, srcp, dstp2, ewp, prevp):
    mesh = plsc.VectorSubcoreMesh(core_axis_name="c", subcore_axis_name="s")
    bpt = BATCHES

    @functools.partial(
        pl.kernel,
        out_type=jax.ShapeDtypeStruct((2 * NP, D), jnp.float32),
        mesh=mesh,
        scratch_types=[
            pltpu.VMEM_SHARED((NP, D), jnp.float32),
            [pltpu.VMEM((CB * B,), jnp.int32)] * 2,
            [pltpu.VMEM((CB, B), jnp.int32)] * 2,
            [pltpu.VMEM((CB * B + 16,), jnp.float32)] * 2,
            [pltpu.VMEM((B, D), jnp.float32)] * NBUF,
            [pltpu.SemaphoreType.DMA] * NBUF,
            [pltpu.SemaphoreType.DMA] * 2,
        ],
    )
    def k(xc_hbm, src_hbm, dst_hbm, ew_hbm, prev_hbm, h_hbm,
          s_acc, idx_b, dst_b, ew_b, rows, sems, ssems):
        idx_a = idx_b[0]
        c = lax.axis_index("c")
        s = lax.axis_index("s")
        coff = c * NP

        def scale_rows(rbuf, ew_a, e_lo):
            @plsc.parallel_loop(0, B, 1, unroll=4)
            def _(j):
                wv = ew_a[pl.ds(e_lo + j, 16)]
                w16 = jnp.full((16,), wv[0])
                for k8 in range(D // 16):
                    sl = (j, pl.ds(k8 * 16, 16))
                    rbuf[sl] = rbuf[sl] * w16

        pltpu.sync_copy(prev_hbm.at[pl.ds(s * ROWS_PER_TILE, ROWS_PER_TILE)],
                        idx_a.at[pl.ds(0, ROWS_PER_TILE)])

        def add_prev_off(j, _):
            sl = pl.ds(j * 16, 16)
            idx_a[sl] = idx_a[sl] + coff
            return 0

        lax.fori_loop(0, ROWS_PER_TILE // 16, add_prev_off, 0)
        row_base = s * ROWS_PER_TILE
        g1 = [None] * ROW_CHUNKS
        for j in range(ROW_CHUNKS):
            g1[j] = pltpu.async_copy(xc_hbm.at[idx_a.at[pl.ds(j * B, B)]],
                                     rows[j % NBUF], sems[j % NBUF])
            if j >= 1:
                g1[j - 1].wait()
                pltpu.sync_copy(rows[(j - 1) % NBUF],
                                s_acc.at[pl.ds(row_base + (j - 1) * B, B)])
        g1[ROW_CHUNKS - 1].wait()
        pltpu.sync_copy(rows[(ROW_CHUNKS - 1) % NBUF],
                        s_acc.at[pl.ds(row_base + (ROW_CHUNKS - 1) * B, B)])
        plsc.subcore_barrier()

        NCH = bpt // CB

        def stage(ch, st):
            e_base = s * EDGES_PER_TILE + ch * (CB * B)
            pltpu.async_copy(src_hbm.at[pl.ds(e_base, CB * B)], idx_b[st],
                             ssems[st])
            pltpu.async_copy(dst_hbm.at[pl.ds(s * bpt + ch * CB, CB), :],
                             dst_b[st], ssems[st])
            pltpu.async_copy(ew_hbm.at[pl.ds(e_base, CB * B)],
                             ew_b[st].at[pl.ds(0, CB * B)], ssems[st])

        def stage_wait(ch, st):
            e_base = s * EDGES_PER_TILE + ch * (CB * B)
            pltpu.make_async_copy(src_hbm.at[pl.ds(e_base, CB * B)], idx_b[st],
                                  ssems[st]).wait()
            pltpu.make_async_copy(dst_hbm.at[pl.ds(s * bpt + ch * CB, CB), :],
                                  dst_b[st], ssems[st]).wait()
            pltpu.make_async_copy(ew_hbm.at[pl.ds(e_base, CB * B)],
                                  ew_b[st].at[pl.ds(0, CB * B)],
                                  ssems[st]).wait()

        def process_chunk(ch, st):
            stage_wait(ch, st)

            @plsc.parallel_loop(0, CB * B // 16, 1, unroll=4)
            def _(b2):
                sl = pl.ds(b2 * 16, 16)
                idx_b[st][sl] = idx_b[st][sl] + coff

            gd = [None] * CB
            sd = [None] * CB

            def do_batch(b):
                buf = b % NBUF
                gd[b].wait()
                scale_rows(rows[buf], ew_b[st], b * B)
                sd[b] = pltpu.async_copy(rows[buf], s_acc.at[dst_b[st].at[b]],
                                         sems[buf], add=True)

            for b in range(CB):
                buf = b % NBUF
                if b >= NBUF:
                    sd[b - NBUF].wait()
                gd[b] = pltpu.async_copy(
                    xc_hbm.at[idx_b[st].at[pl.ds(b * B, B)]], rows[buf],
                    sems[buf])
                if b >= 1:
                    do_batch(b - 1)
            do_batch(CB - 1)
            for t in range(max(0, CB - NBUF), CB):
                sd[t].wait()

        stage(0, 0)

        def chunk_pair(g, _):
            stage(2 * g + 1, 1)
            process_chunk(2 * g, 0)

            @pl.when(g < (NCH - 1) // 2)
            def _():
                stage(2 * g + 2, 0)

            process_chunk(2 * g + 1, 1)
            return 0

        lax.fori_loop(0, NCH // 2, chunk_pair, 0)
        if NCH % 2 == 1:
            process_chunk(NCH - 1, 0)
        plsc.subcore_barrier()

        rd = [None] * ROW_CHUNKS
        wr = [None] * ROW_CHUNKS
        for j in range(ROW_CHUNKS):
            buf = j % NBUF
            if j >= NBUF:
                wr[j - NBUF].wait()
            rd[j] = pltpu.async_copy(s_acc.at[pl.ds(row_base + j * B, B)],
                                     rows[buf], sems[buf])
            rd[j].wait()
            wr[j] = pltpu.async_copy(
                rows[buf], h_hbm.at[pl.ds(coff + row_base + j * B, B)],
                sems[buf])
        for j in range(ROW_CHUNKS - NBUF, ROW_CHUNKS):
            wr[j].wait()

    return k(xc, srcp, dstp2, ewp, prevp)


def _tc_head(h, fs_w2, w2, b2, w_out, b_out):
    blk = 1280
    grid = NP // blk
    hi_blocks = NP // blk

    def body(hlo_ref, hhi_ref, fsw_ref, w2_ref, b2_ref, wout_ref, bout_ref, out_ref):
        g = jax.nn.sigmoid(fsw_ref[...]) * 0.5
        hlo = hlo_ref[...] * g[:, :D]
        hhi = hhi_ref[...] * g[:, D:]
        acc = (
            jnp.dot(hlo, w2_ref[:D, :], preferred_element_type=jnp.float32)
            + jnp.dot(hhi, w2_ref[D:, :], preferred_element_type=jnp.float32)
            + b2_ref[...]
        )
        x = jnp.maximum(acc, 0.0)
        out_ref[...] = (
            jnp.dot(x, wout_ref[...], preferred_element_type=jnp.float32)
            + bout_ref[...]
        )

    return pl.pallas_call(
        body,
        grid=(grid,),
        in_specs=[
            pl.BlockSpec((blk, D), lambda i: (i, 0)),
            pl.BlockSpec((blk, D), lambda i: (i + hi_blocks, 0)),
            pl.BlockSpec((1, 2 * D), lambda i: (0, 0)),
            pl.BlockSpec((2 * D, D), lambda i: (0, 0)),
            pl.BlockSpec((1, D), lambda i: (0, 0)),
            pl.BlockSpec((D, 40), lambda i: (0, 0)),
            pl.BlockSpec((1, 40), lambda i: (0, 0)),
        ],
        out_specs=pl.BlockSpec((blk, 40), lambda i: (i, 0)),
        out_shape=jax.ShapeDtypeStruct((NP, 40), jnp.float32),
    )(h, h, fs_w2, w2, b2, w_out, b_out)


def kernel(X, edge_index, edge_weight, previous_indices, sampled_nodes,
           fs_w, W1, b1, W2, b2, W_out, b_out):
    src = edge_index[2, 0]
    dst = edge_index[2, 1]
    ew = edge_weight[2]
    prev = previous_indices[2]

    pad_rows = NP - N
    xlo = jnp.pad(X[:, :D], ((0, pad_rows), (0, 0)))
    xhi = jnp.pad(X[:, D:], ((0, pad_rows), (0, 0)))
    xc = jnp.concatenate([xlo, xhi], axis=0)
    srcp = jnp.pad(src, (0, EP - E))
    dstp2 = jnp.pad(dst, (0, EP - E)).reshape(EP // B, B)
    ewp = jnp.pad(ew, (0, EP - E))
    prevp = jnp.pad(prev, (0, pad_rows))

    h = _sc_spmm(xc, srcp, dstp2, ewp, prevp)

    out = _tc_head(h, fs_w.reshape(1, 2 * D), W2, b2.reshape(1, -1),
                   W_out, b_out.reshape(1, -1))
    return out[:N]
